# baseline (speedup 1.0000x reference)
import jax
import jax.numpy as jnp
from jax.experimental import pallas as pl
from jax.experimental.pallas import tpu as pltpu

E = 64
D = 1024
FF = 1024
N = 2048
CAP = 40
TN = 512


def _router_body(x_ref, gw_ref, out_ref):
    logits = jax.lax.dot_general(
        x_ref[...], gw_ref[...], (((1,), (1,)), ((), ())),
        preferred_element_type=jnp.float32)
    out_ref[0, :] = jnp.argmax(logits, axis=1).astype(jnp.int32)


def _moe_body(eids_ref, x_ref, wg_ref, wu_ref, wo_ref, y_ref,
              xg_scr, acc_scr, idx_scr, cnt_scr):
    e = pl.program_id(0)

    def gather(j, base):
        cnt = jnp.minimum(cnt_scr[j], CAP)

        def gbody(c, _):
            src = jnp.where(c < cnt, idx_scr[j, c], 0)
            xg_scr[pl.ds(base + c, 1), :] = x_ref[pl.ds(src, 1), :]
            return 0
        jax.lax.fori_loop(0, CAP, gbody, 0, unroll=CAP)

    def scatter(j, base):
        cnt = jnp.minimum(cnt_scr[j], CAP)

        def sbody(c, _):
            dst = jnp.where(c < cnt, idx_scr[j, c], N)
            y_ref[pl.ds(dst, 1), :] = acc_scr[pl.ds(base + c, 1), :]
            return 0
        jax.lax.fori_loop(0, CAP, sbody, 0, unroll=CAP)

    @pl.when(e == 0)
    def _prologue():
        y_ref[...] = jnp.zeros_like(y_ref)

        def zero_body(i, _):
            cnt_scr[i] = 0
            return 0
        jax.lax.fori_loop(0, E, zero_body, 0)

        def scan_body(t, _):
            ee = eids_ref[t]
            p = cnt_scr[ee]
            idx_scr[ee, jnp.minimum(p, CAP)] = t
            cnt_scr[ee] = p + 1
            return 0
        jax.lax.fori_loop(0, N, scan_body, 0, unroll=16)

        gather(0, 0)

    cur = (e % 2) * CAP
    nxt = ((e + 1) % 2) * CAP

    xg = xg_scr[pl.ds(cur, CAP), :]
    g = jax.lax.dot_general(xg, wg_ref[0], (((1,), (1,)), ((), ())),
                            preferred_element_type=jnp.float32)
    u = jax.lax.dot_general(xg, wu_ref[0], (((1,), (1,)), ((), ())),
                            preferred_element_type=jnp.float32)

    @pl.when(e + 1 < E)
    def _prefetch_gather():
        gather(e + 1, nxt)

    @pl.when(e > 0)
    def _deferred_scatter():
        scatter(e - 1, nxt)

    h = (g * jax.nn.sigmoid(g)) * u
    part = jax.lax.dot_general(h, wo_ref[0], (((1,), (1,)), ((), ())),
                               preferred_element_type=jnp.float32)
    acc_scr[pl.ds(cur, CAP), :] = part

    @pl.when(e == E - 1)
    def _final_scatter():
        scatter(e, cur)


def kernel(x, gate_w, wi_gate, wi_up, wo):
    B, S, D_ = x.shape
    xf = x.reshape(N, D)

    eids2d = pl.pallas_call(
        _router_body,
        grid=(N // TN,),
        in_specs=[
            pl.BlockSpec((TN, D), lambda i: (i, 0)),
            pl.BlockSpec((E, D), lambda i: (0, 0)),
        ],
        out_specs=pl.BlockSpec((1, TN), lambda i: (0, i)),
        out_shape=jax.ShapeDtypeStruct((1, N), jnp.int32),
    )(xf, gate_w)
    eids = eids2d.reshape(N)

    ypad = pl.pallas_call(
        _moe_body,
        grid_spec=pltpu.PrefetchScalarGridSpec(
            num_scalar_prefetch=1,
            grid=(E,),
            in_specs=[
                pl.BlockSpec((N, D), lambda e, sref: (0, 0)),
                pl.BlockSpec((1, FF, D), lambda e, sref: (e, 0, 0)),
                pl.BlockSpec((1, FF, D), lambda e, sref: (e, 0, 0)),
                pl.BlockSpec((1, D, FF), lambda e, sref: (e, 0, 0)),
            ],
            out_specs=pl.BlockSpec((N + 8, D), lambda e, sref: (0, 0)),
            scratch_shapes=[
                pltpu.VMEM((2 * CAP, D), jnp.float32),
                pltpu.VMEM((2 * CAP, D), jnp.float32),
                pltpu.SMEM((E, CAP + 1), jnp.int32),
                pltpu.SMEM((E,), jnp.int32),
            ],
        ),
        out_shape=jax.ShapeDtypeStruct((N + 8, D), jnp.float32),
    )(eids, xf, wi_gate, wi_up, wo)

    return ypad[:N].reshape(B, S, D_)


# speedup vs baseline: 1.0217x; 1.0217x over previous
import jax
import jax.numpy as jnp
from jax.experimental import pallas as pl
from jax.experimental.pallas import tpu as pltpu

E = 64
D = 1024
FF = 1024
N = 2048
CAP = 40
FT = 1024
NF = FF // FT
TN = 512


def _router_body(x_ref, gw_ref, out_ref):
    logits = jax.lax.dot_general(
        x_ref[...], gw_ref[...], (((1,), (1,)), ((), ())),
        preferred_element_type=jnp.float32)
    out_ref[0, :] = jnp.argmax(logits, axis=1).astype(jnp.int32)


def _moe_body(eids_ref, x_ref, wg_ref, wu_ref, wo_ref, y_ref,
              xg_scr, acc_scr, idx_scr, cnt_scr):
    e = pl.program_id(0)
    f = pl.program_id(1)

    @pl.when(jnp.logical_and(e == 0, f == 0))
    def _prologue():
        y_ref[...] = jnp.zeros_like(y_ref)

        def zero_body(i, _):
            cnt_scr[i] = 0
            return 0
        jax.lax.fori_loop(0, E, zero_body, 0)

        def scan_body(t, _):
            ee = eids_ref[t]
            p = cnt_scr[ee]
            idx_scr[ee, jnp.minimum(p, CAP)] = t
            cnt_scr[ee] = p + 1
            return 0
        jax.lax.fori_loop(0, N, scan_body, 0, unroll=16)

    @pl.when(f == 0)
    def _gather():
        cnt = jnp.minimum(cnt_scr[e], CAP)

        def gbody(c, _):
            src = jnp.where(c < cnt, idx_scr[e, c], 0)
            xg_scr[pl.ds(c, 1), :] = x_ref[pl.ds(src, 1), :]
            return 0
        jax.lax.fori_loop(0, CAP, gbody, 0, unroll=CAP)

    xg = xg_scr[...]
    g = jax.lax.dot_general(xg, wg_ref[0], (((1,), (1,)), ((), ())),
                            preferred_element_type=jnp.float32)
    u = jax.lax.dot_general(xg, wu_ref[0], (((1,), (1,)), ((), ())),
                            preferred_element_type=jnp.float32)
    h = (g * jax.nn.sigmoid(g)) * u
    part = jax.lax.dot_general(h, wo_ref[0], (((1,), (1,)), ((), ())),
                               preferred_element_type=jnp.float32)

    @pl.when(f == 0)
    def _init_acc():
        acc_scr[...] = part

    @pl.when(f > 0)
    def _add_acc():
        acc_scr[...] += part

    @pl.when(f == NF - 1)
    def _scatter():
        cnt = jnp.minimum(cnt_scr[e], CAP)

        def sbody(c, _):
            dst = jnp.where(c < cnt, idx_scr[e, c], N)
            y_ref[pl.ds(dst, 1), :] = acc_scr[pl.ds(c, 1), :]
            return 0
        jax.lax.fori_loop(0, CAP, sbody, 0, unroll=CAP)


def kernel(x, gate_w, wi_gate, wi_up, wo):
    B, S, D_ = x.shape
    xf = x.reshape(N, D)

    eids2d = pl.pallas_call(
        _router_body,
        grid=(N // TN,),
        in_specs=[
            pl.BlockSpec((TN, D), lambda i: (i, 0)),
            pl.BlockSpec((E, D), lambda i: (0, 0)),
        ],
        out_specs=pl.BlockSpec((1, TN), lambda i: (0, i)),
        out_shape=jax.ShapeDtypeStruct((1, N), jnp.int32),
    )(xf, gate_w)
    eids = eids2d.reshape(N)

    ypad = pl.pallas_call(
        _moe_body,
        grid_spec=pltpu.PrefetchScalarGridSpec(
            num_scalar_prefetch=1,
            grid=(E, NF),
            in_specs=[
                pl.BlockSpec((N, D), lambda e, f, sref: (0, 0)),
                pl.BlockSpec((1, FT, D), lambda e, f, sref: (e, f, 0)),
                pl.BlockSpec((1, FT, D), lambda e, f, sref: (e, f, 0)),
                pl.BlockSpec((1, D, FT), lambda e, f, sref: (e, 0, f)),
            ],
            out_specs=pl.BlockSpec((N + 8, D), lambda e, f, sref: (0, 0)),
            scratch_shapes=[
                pltpu.VMEM((CAP, D), jnp.float32),
                pltpu.VMEM((CAP, D), jnp.float32),
                pltpu.SMEM((E, CAP + 1), jnp.int32),
                pltpu.SMEM((E,), jnp.int32),
            ],
        ),
        out_shape=jax.ShapeDtypeStruct((N + 8, D), jnp.float32),
    )(eids, xf, wi_gate, wi_up, wo)

    return ypad[:N].reshape(B, S, D_)
